# baseline (speedup 1.0000x reference)
import functools

import jax
import jax.numpy as jnp
from jax import lax
from jax.experimental import pallas as pl
from jax.experimental.pallas import tpu as pltpu
from jax.experimental.pallas import tpu_sc as plsc

N_USER = 100000
M_ITEM = 100000
DIM = 64
BATCH = 4096

_L = 16
_NC = 2
_NS = 16
_NW = _NC * _NS
_BPW = BATCH // _NW
_VPR = DIM // _L


def _lane_shuffle(x, idx):
    return lax.gather(
        x,
        idx.reshape(_L, 1),
        lax.GatherDimensionNumbers(
            offset_dims=(), collapsed_slice_dims=(0,), start_index_map=(0,)),
        slice_sizes=(1,),
        mode=lax.GatherScatterMode.PROMISE_IN_BOUNDS,
    )


def _sc_scores_kernel(user_hbm, item_hbm, uid_hbm, iid_hbm, out_hbm,
                      uidx_v, iidx_v, urows_v, irows_v, out_v, sem_u, sem_i):
    wid = lax.axis_index("s") * _NC + lax.axis_index("c")
    base = wid * _BPW

    pltpu.sync_copy(uid_hbm.at[pl.ds(base, _BPW)], uidx_v)
    pltpu.sync_copy(iid_hbm.at[pl.ds(base, _BPW)], iidx_v)

    cp_u = pltpu.async_copy(user_hbm.at[uidx_v], urows_v, sem_u)
    cp_i = pltpu.async_copy(item_hbm.at[iidx_v], irows_v, sem_i)
    cp_u.wait()
    cp_i.wait()

    lane = lax.iota(jnp.int32, _L)
    perms = [lane ^ k for k in (1, 2, 4, 8)]

    def group_body(g, carry):
        out_vec = jnp.zeros((_L,), jnp.float32)
        for r in range(_L):
            b = g * _L + r
            acc = urows_v[b, pl.ds(0, _L)] * irows_v[b, pl.ds(0, _L)]
            for j in range(1, _VPR):
                acc = acc + urows_v[b, pl.ds(j * _L, _L)] * irows_v[b, pl.ds(j * _L, _L)]
            for p in perms:
                acc = acc + _lane_shuffle(acc, p)
            out_vec = jnp.where(lane == r, acc, out_vec)
        out_v[pl.ds(g * _L, _L)] = out_vec
        return carry

    lax.fori_loop(0, _BPW // _L, group_body, 0)

    pltpu.sync_copy(out_v, out_hbm.at[pl.ds(base, _BPW)])


@jax.jit
def kernel(user_table, item_table, user_ids, item_ids):
    mesh = plsc.VectorSubcoreMesh(core_axis_name="c", subcore_axis_name="s")
    run = functools.partial(
        pl.kernel,
        mesh=mesh,
        out_type=jax.ShapeDtypeStruct((BATCH,), jnp.float32),
        scratch_types=[
            pltpu.VMEM((_BPW,), jnp.int32),
            pltpu.VMEM((_BPW,), jnp.int32),
            pltpu.VMEM((_BPW, DIM), jnp.float32),
            pltpu.VMEM((_BPW, DIM), jnp.float32),
            pltpu.VMEM((_BPW,), jnp.float32),
            pltpu.SemaphoreType.DMA,
            pltpu.SemaphoreType.DMA,
        ],
        compiler_params=pltpu.CompilerParams(use_tc_tiling_on_sc=False),
    )(_sc_scores_kernel)
    return run(user_table, item_table,
               user_ids.astype(jnp.int32), item_ids.astype(jnp.int32))
